# Initial kernel scaffold; baseline (speedup 1.0000x reference)
import jax
import jax.numpy as jnp
from jax.experimental import pallas as pl


def kernel(x, pos_edge_index, neg_edge_index, c1_Wpl, c1_Wpr, c1_bpr, c1_Wnl, c1_Wnr, c1_bnr, c2_Wpl, c2_Wpr, c2_bpr, c2_Wnl, c2_Wnr, c2_bnr):
    raise NotImplementedError("write your pallas kernel here")


# speedup vs baseline: 11.9318x; 11.9318x over previous
import functools

import jax
import jax.numpy as jnp
from jax import lax
from jax.experimental import pallas as pl
from jax.experimental.pallas import tpu as pltpu
from jax.experimental.pallas import tpu_sc as plsc

N = 10000
NP = 10240
E = 160000
NT = 16
CH = 80
B = 125
RPT = NP // NT

_f32 = jnp.float32
_mesh = plsc.VectorSubcoreMesh(core_axis_name="c", subcore_axis_name="s")


def _sc_agg_counts_body(SRC, DST, T, Z32, Z8, ONES, outS, outC,
                        idx_s, idx_d, rows, ones_v, acc, accc, sem):
    c = lax.axis_index("c")
    s = lax.axis_index("s")
    row0 = c * (NT * CH) + s * CH
    pltpu.sync_copy(SRC.at[pl.ds(row0, CH)], idx_s)
    pltpu.sync_copy(DST.at[pl.ds(row0, CH)], idx_d)
    pltpu.sync_copy(ONES, ones_v)
    r0 = s * RPT
    pltpu.sync_copy(Z32.at[pl.ds(r0, RPT)], acc.at[pl.ds(r0, RPT)])
    pltpu.sync_copy(Z8.at[pl.ds(r0, RPT)], accc.at[pl.ds(r0, RPT)])
    plsc.subcore_barrier()

    def chunk(j, carry):
        pltpu.async_copy(T.at[idx_s.at[j]], rows, sem).wait()
        pltpu.sync_copy(rows, acc.at[idx_d.at[j]], add=True)
        pltpu.sync_copy(ones_v, accc.at[idx_d.at[j]], add=True)
        return carry

    lax.fori_loop(0, CH, chunk, 0)
    plsc.subcore_barrier()
    o0 = c * NP + r0
    pltpu.sync_copy(acc.at[pl.ds(r0, RPT)], outS.at[pl.ds(o0, RPT)])
    pltpu.sync_copy(accc.at[pl.ds(r0, RPT)], outC.at[pl.ds(o0, RPT)])


_sc_layer1 = functools.partial(
    pl.kernel,
    out_type=(jax.ShapeDtypeStruct((2 * NP, 32), _f32),
              jax.ShapeDtypeStruct((2 * NP, 8), _f32)),
    mesh=_mesh,
    scratch_types=(pltpu.VMEM((CH, B), jnp.int32),
                   pltpu.VMEM((CH, B), jnp.int32),
                   pltpu.VMEM((B, 32), _f32),
                   pltpu.VMEM((B, 8), _f32),
                   pltpu.VMEM_SHARED((NP, 32), _f32),
                   pltpu.VMEM_SHARED((NP, 8), _f32),
                   pltpu.SemaphoreType.DMA),
    compiler_params=pltpu.CompilerParams(use_tc_tiling_on_sc=False),
)(_sc_agg_counts_body)


def _sc_agg_body(SRC, DST, T, Z64, outS, idx_s, idx_d, rows, acc, sem):
    c = lax.axis_index("c")
    s = lax.axis_index("s")
    row0 = c * (NT * CH) + s * CH
    pltpu.sync_copy(SRC.at[pl.ds(row0, CH)], idx_s)
    pltpu.sync_copy(DST.at[pl.ds(row0, CH)], idx_d)
    r0 = s * RPT
    pltpu.sync_copy(Z64.at[pl.ds(r0, RPT)], acc.at[pl.ds(r0, RPT)])
    plsc.subcore_barrier()

    def chunk(j, carry):
        pltpu.async_copy(T.at[idx_s.at[j]], rows, sem).wait()
        pltpu.sync_copy(rows, acc.at[idx_d.at[j]], add=True)
        return carry

    lax.fori_loop(0, CH, chunk, 0)
    plsc.subcore_barrier()
    o0 = c * NP + r0
    pltpu.sync_copy(acc.at[pl.ds(r0, RPT)], outS.at[pl.ds(o0, RPT)])


_sc_layer2 = functools.partial(
    pl.kernel,
    out_type=jax.ShapeDtypeStruct((2 * NP, 64), _f32),
    mesh=_mesh,
    scratch_types=(pltpu.VMEM((CH, B), jnp.int32),
                   pltpu.VMEM((CH, B), jnp.int32),
                   pltpu.VMEM((B, 64), _f32),
                   pltpu.VMEM_SHARED((NP, 64), _f32),
                   pltpu.SemaphoreType.DMA),
    compiler_params=pltpu.CompilerParams(use_tc_tiling_on_sc=False),
)(_sc_agg_body)


def _tc1_body(x_ref, w_ref, b_ref, t_ref, r_ref):
    m = jnp.dot(x_ref[...], w_ref[...], preferred_element_type=_f32)
    t_ref[0:N, :] = m[:, 0:32]
    t_ref[N:2 * N, :] = m[:, 32:64]
    r_ref[...] = m[:, 64:128] + b_ref[...]


def _tc2_body(s1_ref, c_ref, r1_ref, wp_ref, wn_ref, b_ref, u_ref, r2_ref):
    cp = jnp.maximum(c_ref[0:N, 0:1], 1.0)
    cn = jnp.maximum(c_ref[NP:NP + N, 0:1], 1.0)
    zp = jnp.maximum(r1_ref[:, 0:32] + s1_ref[0:N, :] / cp, 0.0)
    zn = jnp.maximum(r1_ref[:, 32:64] + s1_ref[NP:NP + N, :] / cn, 0.0)
    p = jnp.dot(zp, wp_ref[...], preferred_element_type=_f32)
    q = jnp.dot(zn, wn_ref[...], preferred_element_type=_f32)
    u_ref[0:N, 0:32] = p[:, 0:32]
    u_ref[0:N, 32:64] = q[:, 0:32]
    u_ref[N:2 * N, 0:32] = q[:, 32:64]
    u_ref[N:2 * N, 32:64] = p[:, 32:64]
    r2_ref[:, 0:32] = p[:, 64:96] + b_ref[:, 0:32]
    r2_ref[:, 32:64] = q[:, 64:96] + b_ref[:, 32:64]


def _tc3_body(s2_ref, c_ref, r2_ref, out_ref):
    cp = jnp.maximum(c_ref[0:N, 0:1], 1.0)
    cn = jnp.maximum(c_ref[NP:NP + N, 0:1], 1.0)
    ap = s2_ref[0:N, :] / cp
    an = s2_ref[NP:NP + N, :] / cn
    out_ref[...] = jnp.maximum(r2_ref[...] + ap + an, 0.0)


def kernel(x, pos_edge_index, neg_edge_index,
           c1_Wpl, c1_Wpr, c1_bpr, c1_Wnl, c1_Wnr, c1_bnr,
           c2_Wpl, c2_Wpr, c2_bpr, c2_Wnl, c2_Wnr, c2_bnr):
    w1 = jnp.concatenate([c1_Wpl.T, c1_Wnl.T, c1_Wpr.T, c1_Wnr.T], axis=1)
    b1 = jnp.concatenate([c1_bpr, c1_bnr]).reshape(1, 64)
    wp = jnp.concatenate([c2_Wpl[:, :32].T, c2_Wnl[:, 32:].T, c2_Wpr.T], axis=1)
    wn = jnp.concatenate([c2_Wnl[:, :32].T, c2_Wpl[:, 32:].T, c2_Wnr.T], axis=1)
    b2 = jnp.concatenate([c2_bpr, c2_bnr]).reshape(1, 64)

    src = jnp.concatenate([pos_edge_index[0].reshape(NT * CH, B),
                           neg_edge_index[0].reshape(NT * CH, B) + N], axis=0)
    dst = jnp.concatenate([pos_edge_index[1].reshape(NT * CH, B),
                           neg_edge_index[1].reshape(NT * CH, B)], axis=0)

    z32 = jnp.zeros((NP, 32), _f32)
    z8 = jnp.zeros((NP, 8), _f32)
    z64 = jnp.zeros((NP, 64), _f32)
    ones = jnp.ones((B, 8), _f32)

    t1, r1 = pl.pallas_call(
        _tc1_body,
        out_shape=[jax.ShapeDtypeStruct((2 * N, 32), _f32),
                   jax.ShapeDtypeStruct((N, 64), _f32)],
    )(x, w1, b1)

    s1, cnt = _sc_layer1(src, dst, t1, z32, z8, ones)

    u, r2 = pl.pallas_call(
        _tc2_body,
        out_shape=[jax.ShapeDtypeStruct((2 * N, 64), _f32),
                   jax.ShapeDtypeStruct((N, 64), _f32)],
    )(s1, cnt, r1, wp, wn, b2)

    s2 = _sc_layer2(src, dst, u, z64)

    out = pl.pallas_call(
        _tc3_body,
        out_shape=jax.ShapeDtypeStruct((N, 64), _f32),
    )(s2, cnt, r2)
    return out


# speedup vs baseline: 18.5777x; 1.5570x over previous
import functools

import jax
import jax.numpy as jnp
from jax import lax
from jax.experimental import pallas as pl
from jax.experimental.pallas import tpu as pltpu
from jax.experimental.pallas import tpu_sc as plsc

N = 10000
NP = 10240
E = 160000
NT = 16
CH = 80
B = 125
RPT = NP // NT

_f32 = jnp.float32
_mesh = plsc.VectorSubcoreMesh(core_axis_name="c", subcore_axis_name="s")


NBUF = 4


def _sc_agg_counts_body(SRC, DST, T, Z32, Z8, ONES, outS, outC,
                        idx_s, idx_d, rows, ones_v, acc, accc, sems):
    c = lax.axis_index("c")
    s = lax.axis_index("s")
    row0 = c * (NT * CH) + s * CH
    pltpu.sync_copy(SRC.at[pl.ds(row0, CH)], idx_s)
    pltpu.sync_copy(DST.at[pl.ds(row0, CH)], idx_d)
    pltpu.sync_copy(ONES, ones_v)
    r0 = s * RPT
    pltpu.sync_copy(Z32.at[pl.ds(r0, RPT)], acc.at[pl.ds(r0, RPT)])
    pltpu.sync_copy(Z8.at[pl.ds(r0, RPT)], accc.at[pl.ds(r0, RPT)])
    plsc.subcore_barrier()

    for b in range(NBUF):
        pltpu.async_copy(T.at[idx_s.at[b]], rows.at[b], sems.at[b])

    def group(g, carry):
        for b in range(NBUF):
            j = g * NBUF + b
            pltpu.make_async_copy(T.at[idx_s.at[j]], rows.at[b],
                                  sems.at[b]).wait()
            pltpu.sync_copy(rows.at[b], acc.at[idx_d.at[j]], add=True)
            nxt = j + NBUF

            @pl.when(nxt < CH)
            def _():
                pltpu.async_copy(T.at[idx_s.at[nxt]], rows.at[b], sems.at[b])

            pltpu.sync_copy(ones_v, accc.at[idx_d.at[j]], add=True)
        return carry

    lax.fori_loop(0, CH // NBUF, group, 0)
    plsc.subcore_barrier()
    o0 = c * NP + r0
    pltpu.sync_copy(acc.at[pl.ds(r0, RPT)], outS.at[pl.ds(o0, RPT)])
    pltpu.sync_copy(accc.at[pl.ds(r0, RPT)], outC.at[pl.ds(o0, RPT)])


_sc_layer1 = functools.partial(
    pl.kernel,
    out_type=(jax.ShapeDtypeStruct((2 * NP, 32), _f32),
              jax.ShapeDtypeStruct((2 * NP, 8), _f32)),
    mesh=_mesh,
    scratch_types=(pltpu.VMEM((CH, B), jnp.int32),
                   pltpu.VMEM((CH, B), jnp.int32),
                   pltpu.VMEM((NBUF, B, 32), _f32),
                   pltpu.VMEM((B, 8), _f32),
                   pltpu.VMEM_SHARED((NP, 32), _f32),
                   pltpu.VMEM_SHARED((NP, 8), _f32),
                   pltpu.SemaphoreType.DMA((NBUF,))),
    compiler_params=pltpu.CompilerParams(use_tc_tiling_on_sc=False),
)(_sc_agg_counts_body)


def _sc_agg_body(SRC, DST, T, Z64, outS, idx_s, idx_d, rows, acc, sems):
    c = lax.axis_index("c")
    s = lax.axis_index("s")
    row0 = c * (NT * CH) + s * CH
    pltpu.sync_copy(SRC.at[pl.ds(row0, CH)], idx_s)
    pltpu.sync_copy(DST.at[pl.ds(row0, CH)], idx_d)
    r0 = s * RPT
    pltpu.sync_copy(Z64.at[pl.ds(r0, RPT)], acc.at[pl.ds(r0, RPT)])
    plsc.subcore_barrier()

    for b in range(NBUF):
        pltpu.async_copy(T.at[idx_s.at[b]], rows.at[b], sems.at[b])

    def group(g, carry):
        for b in range(NBUF):
            j = g * NBUF + b
            pltpu.make_async_copy(T.at[idx_s.at[j]], rows.at[b],
                                  sems.at[b]).wait()
            pltpu.sync_copy(rows.at[b], acc.at[idx_d.at[j]], add=True)
            nxt = j + NBUF

            @pl.when(nxt < CH)
            def _():
                pltpu.async_copy(T.at[idx_s.at[nxt]], rows.at[b], sems.at[b])

        return carry

    lax.fori_loop(0, CH // NBUF, group, 0)
    plsc.subcore_barrier()
    o0 = c * NP + r0
    pltpu.sync_copy(acc.at[pl.ds(r0, RPT)], outS.at[pl.ds(o0, RPT)])


_sc_layer2 = functools.partial(
    pl.kernel,
    out_type=jax.ShapeDtypeStruct((2 * NP, 64), _f32),
    mesh=_mesh,
    scratch_types=(pltpu.VMEM((CH, B), jnp.int32),
                   pltpu.VMEM((CH, B), jnp.int32),
                   pltpu.VMEM((NBUF, B, 64), _f32),
                   pltpu.VMEM_SHARED((NP, 64), _f32),
                   pltpu.SemaphoreType.DMA((NBUF,))),
    compiler_params=pltpu.CompilerParams(use_tc_tiling_on_sc=False),
)(_sc_agg_body)


def _tc1_body(x_ref, w_ref, b_ref, t_ref, r_ref):
    m = jnp.dot(x_ref[...], w_ref[...], preferred_element_type=_f32)
    t_ref[0:N, :] = m[:, 0:32]
    t_ref[N:2 * N, :] = m[:, 32:64]
    r_ref[...] = m[:, 64:128] + b_ref[...]


def _tc2_body(s1_ref, c_ref, r1_ref, wp_ref, wn_ref, b_ref, u_ref, r2_ref):
    cp = jnp.maximum(c_ref[0:N, 0:1], 1.0)
    cn = jnp.maximum(c_ref[NP:NP + N, 0:1], 1.0)
    zp = jnp.maximum(r1_ref[:, 0:32] + s1_ref[0:N, :] / cp, 0.0)
    zn = jnp.maximum(r1_ref[:, 32:64] + s1_ref[NP:NP + N, :] / cn, 0.0)
    p = jnp.dot(zp, wp_ref[...], preferred_element_type=_f32)
    q = jnp.dot(zn, wn_ref[...], preferred_element_type=_f32)
    u_ref[0:N, 0:32] = p[:, 0:32]
    u_ref[0:N, 32:64] = q[:, 0:32]
    u_ref[N:2 * N, 0:32] = q[:, 32:64]
    u_ref[N:2 * N, 32:64] = p[:, 32:64]
    r2_ref[:, 0:32] = p[:, 64:96] + b_ref[:, 0:32]
    r2_ref[:, 32:64] = q[:, 64:96] + b_ref[:, 32:64]


def _tc3_body(s2_ref, c_ref, r2_ref, out_ref):
    cp = jnp.maximum(c_ref[0:N, 0:1], 1.0)
    cn = jnp.maximum(c_ref[NP:NP + N, 0:1], 1.0)
    ap = s2_ref[0:N, :] / cp
    an = s2_ref[NP:NP + N, :] / cn
    out_ref[...] = jnp.maximum(r2_ref[...] + ap + an, 0.0)


def kernel(x, pos_edge_index, neg_edge_index,
           c1_Wpl, c1_Wpr, c1_bpr, c1_Wnl, c1_Wnr, c1_bnr,
           c2_Wpl, c2_Wpr, c2_bpr, c2_Wnl, c2_Wnr, c2_bnr):
    w1 = jnp.concatenate([c1_Wpl.T, c1_Wnl.T, c1_Wpr.T, c1_Wnr.T], axis=1)
    b1 = jnp.concatenate([c1_bpr, c1_bnr]).reshape(1, 64)
    wp = jnp.concatenate([c2_Wpl[:, :32].T, c2_Wnl[:, 32:].T, c2_Wpr.T], axis=1)
    wn = jnp.concatenate([c2_Wnl[:, :32].T, c2_Wpl[:, 32:].T, c2_Wnr.T], axis=1)
    b2 = jnp.concatenate([c2_bpr, c2_bnr]).reshape(1, 64)

    src = jnp.concatenate([pos_edge_index[0].reshape(NT * CH, B),
                           neg_edge_index[0].reshape(NT * CH, B) + N], axis=0)
    dst = jnp.concatenate([pos_edge_index[1].reshape(NT * CH, B),
                           neg_edge_index[1].reshape(NT * CH, B)], axis=0)

    z32 = jnp.zeros((NP, 32), _f32)
    z8 = jnp.zeros((NP, 8), _f32)
    z64 = jnp.zeros((NP, 64), _f32)
    ones = jnp.ones((B, 8), _f32)

    t1, r1 = pl.pallas_call(
        _tc1_body,
        out_shape=[jax.ShapeDtypeStruct((2 * N, 32), _f32),
                   jax.ShapeDtypeStruct((N, 64), _f32)],
    )(x, w1, b1)

    s1, cnt = _sc_layer1(src, dst, t1, z32, z8, ones)

    u, r2 = pl.pallas_call(
        _tc2_body,
        out_shape=[jax.ShapeDtypeStruct((2 * N, 64), _f32),
                   jax.ShapeDtypeStruct((N, 64), _f32)],
    )(s1, cnt, r1, wp, wn, b2)

    s2 = _sc_layer2(src, dst, u, z64)

    out = pl.pallas_call(
        _tc3_body,
        out_shape=jax.ShapeDtypeStruct((N, 64), _f32),
    )(s2, cnt, r2)
    return out


# speedup vs baseline: 19.6469x; 1.0576x over previous
import functools

import jax
import jax.numpy as jnp
from jax import lax
from jax.experimental import pallas as pl
from jax.experimental.pallas import tpu as pltpu
from jax.experimental.pallas import tpu_sc as plsc

N = 10000
NP = 10240
E = 160000
NT = 16
CH = 80
B = 125
RPT = NP // NT

_f32 = jnp.float32
_mesh = plsc.VectorSubcoreMesh(core_axis_name="c", subcore_axis_name="s")


NBUF = 8
SLACK = 2


def _agg_pipeline(c, s, PE, NE, T, idx_s, idx_d, rows, acc, gsems, ssems,
                  ones_v=None, accc=None, osems=None):
    row0 = s * CH

    @pl.when(c == 0)
    def _():
        pltpu.sync_copy(PE.at[0, pl.ds(row0, CH)], idx_s)
        pltpu.sync_copy(PE.at[1, pl.ds(row0, CH)], idx_d)

    @pl.when(c == 1)
    def _():
        pltpu.sync_copy(NE.at[0, pl.ds(row0, CH)], idx_s)
        pltpu.sync_copy(NE.at[1, pl.ds(row0, CH)], idx_d)

    plsc.subcore_barrier()

    for b in range(NBUF):
        pltpu.async_copy(T.at[idx_s.at[b]], rows.at[b], gsems.at[b])

    def group(g, carry):
        for b in range(NBUF):
            j = g * NBUF + b
            pltpu.make_async_copy(T.at[idx_s.at[j]], rows.at[b],
                                  gsems.at[b]).wait()
            pltpu.async_copy(rows.at[b], acc.at[idx_d.at[j]], ssems.at[b],
                             add=True)
            if accc is not None:
                @pl.when(j >= NBUF)
                def _():
                    pltpu.make_async_copy(ones_v, accc.at[idx_d.at[0]],
                                          osems.at[b]).wait()

                pltpu.async_copy(ones_v, accc.at[idx_d.at[j]], osems.at[b],
                                 add=True)
            r = j + NBUF - SLACK
            rb = (b + NBUF - SLACK) % NBUF

            @pl.when((r >= NBUF) & (r < CH))
            def _():
                pltpu.make_async_copy(rows.at[rb], acc.at[idx_d.at[0]],
                                      ssems.at[rb]).wait()
                pltpu.async_copy(T.at[idx_s.at[r]], rows.at[rb], gsems.at[rb])

        return carry

    lax.fori_loop(0, CH // NBUF, group, 0)
    for b in range(NBUF):
        pltpu.make_async_copy(rows.at[b], acc.at[idx_d.at[0]],
                              ssems.at[b]).wait()
        if accc is not None:
            pltpu.make_async_copy(ones_v, accc.at[idx_d.at[0]],
                                  osems.at[b]).wait()


def _sc_agg_counts_body(PE, NE, T, Z32, Z8, ONES, outS, outC,
                        idx_s, idx_d, rows, ones_v, acc, accc,
                        gsems, ssems, osems):
    c = lax.axis_index("c")
    s = lax.axis_index("s")
    pltpu.sync_copy(ONES, ones_v)
    r0 = s * RPT
    pltpu.sync_copy(Z32.at[pl.ds(r0, RPT)], acc.at[pl.ds(r0, RPT)])
    pltpu.sync_copy(Z8.at[pl.ds(r0, RPT)], accc.at[pl.ds(r0, RPT)])
    _agg_pipeline(c, s, PE, NE, T, idx_s, idx_d, rows, acc, gsems, ssems,
                  ones_v=ones_v, accc=accc, osems=osems)
    plsc.subcore_barrier()
    o0 = c * NP + r0
    pltpu.sync_copy(acc.at[pl.ds(r0, RPT)], outS.at[pl.ds(o0, RPT)])
    pltpu.sync_copy(accc.at[pl.ds(r0, RPT)], outC.at[pl.ds(o0, RPT)])


_sc_layer1 = functools.partial(
    pl.kernel,
    out_type=(jax.ShapeDtypeStruct((2 * NP, 32), _f32),
              jax.ShapeDtypeStruct((2 * NP, 8), _f32)),
    mesh=_mesh,
    scratch_types=(pltpu.VMEM((CH, B), jnp.int32),
                   pltpu.VMEM((CH, B), jnp.int32),
                   pltpu.VMEM((NBUF, B, 32), _f32),
                   pltpu.VMEM((B, 8), _f32),
                   pltpu.VMEM_SHARED((NP, 32), _f32),
                   pltpu.VMEM_SHARED((NP, 8), _f32),
                   pltpu.SemaphoreType.DMA((NBUF,)),
                   pltpu.SemaphoreType.DMA((NBUF,)),
                   pltpu.SemaphoreType.DMA((NBUF,))),
    compiler_params=pltpu.CompilerParams(use_tc_tiling_on_sc=False),
)(_sc_agg_counts_body)


def _sc_agg_body(PE, NE, T, Z64, outS, idx_s, idx_d, rows, acc, gsems, ssems):
    c = lax.axis_index("c")
    s = lax.axis_index("s")
    r0 = s * RPT
    pltpu.sync_copy(Z64.at[pl.ds(r0, RPT)], acc.at[pl.ds(r0, RPT)])
    _agg_pipeline(c, s, PE, NE, T, idx_s, idx_d, rows, acc, gsems, ssems)
    plsc.subcore_barrier()
    o0 = c * NP + r0
    pltpu.sync_copy(acc.at[pl.ds(r0, RPT)], outS.at[pl.ds(o0, RPT)])


_sc_layer2 = functools.partial(
    pl.kernel,
    out_type=jax.ShapeDtypeStruct((2 * NP, 64), _f32),
    mesh=_mesh,
    scratch_types=(pltpu.VMEM((CH, B), jnp.int32),
                   pltpu.VMEM((CH, B), jnp.int32),
                   pltpu.VMEM((NBUF, B, 64), _f32),
                   pltpu.VMEM_SHARED((NP, 64), _f32),
                   pltpu.SemaphoreType.DMA((NBUF,)),
                   pltpu.SemaphoreType.DMA((NBUF,))),
    compiler_params=pltpu.CompilerParams(use_tc_tiling_on_sc=False),
)(_sc_agg_body)


def _tc1_body(x_ref, w_ref, b_ref, t_ref, r_ref):
    m = jnp.dot(x_ref[...], w_ref[...], preferred_element_type=_f32)
    t_ref[0:N, :] = m[:, 0:32]
    t_ref[N:2 * N, :] = m[:, 32:64]
    r_ref[...] = m[:, 64:128] + b_ref[...]


def _tc2_body(s1_ref, c_ref, r1_ref, wp_ref, wn_ref, b_ref, u_ref, r2_ref):
    cp = jnp.maximum(c_ref[0:N, 0:1], 1.0)
    cn = jnp.maximum(c_ref[NP:NP + N, 0:1], 1.0)
    zp = jnp.maximum(r1_ref[:, 0:32] + s1_ref[0:N, :] / cp, 0.0)
    zn = jnp.maximum(r1_ref[:, 32:64] + s1_ref[NP:NP + N, :] / cn, 0.0)
    p = jnp.dot(zp, wp_ref[...], preferred_element_type=_f32)
    q = jnp.dot(zn, wn_ref[...], preferred_element_type=_f32)
    u_ref[0:N, 0:32] = p[:, 0:32]
    u_ref[0:N, 32:64] = q[:, 0:32]
    u_ref[N:2 * N, 0:32] = q[:, 32:64]
    u_ref[N:2 * N, 32:64] = p[:, 32:64]
    r2_ref[:, 0:32] = p[:, 64:96] + b_ref[:, 0:32]
    r2_ref[:, 32:64] = q[:, 64:96] + b_ref[:, 32:64]


def _tc3_body(s2_ref, c_ref, r2_ref, out_ref):
    cp = jnp.maximum(c_ref[0:N, 0:1], 1.0)
    cn = jnp.maximum(c_ref[NP:NP + N, 0:1], 1.0)
    ap = s2_ref[0:N, :] / cp
    an = s2_ref[NP:NP + N, :] / cn
    out_ref[...] = jnp.maximum(r2_ref[...] + ap + an, 0.0)


def kernel(x, pos_edge_index, neg_edge_index,
           c1_Wpl, c1_Wpr, c1_bpr, c1_Wnl, c1_Wnr, c1_bnr,
           c2_Wpl, c2_Wpr, c2_bpr, c2_Wnl, c2_Wnr, c2_bnr):
    w1 = jnp.concatenate([c1_Wpl.T, c1_Wnl.T, c1_Wpr.T, c1_Wnr.T], axis=1)
    b1 = jnp.concatenate([c1_bpr, c1_bnr]).reshape(1, 64)
    wp = jnp.concatenate([c2_Wpl[:, :32].T, c2_Wnl[:, 32:].T, c2_Wpr.T], axis=1)
    wn = jnp.concatenate([c2_Wnl[:, :32].T, c2_Wpl[:, 32:].T, c2_Wnr.T], axis=1)
    b2 = jnp.concatenate([c2_bpr, c2_bnr]).reshape(1, 64)

    pe = pos_edge_index.reshape(2, NT * CH, B)
    ne = (neg_edge_index.reshape(2, NT * CH, B)
          + jnp.array([N, 0], jnp.int32).reshape(2, 1, 1))

    z32 = jnp.zeros((NP, 32), _f32)
    z8 = jnp.zeros((NP, 8), _f32)
    z64 = jnp.zeros((NP, 64), _f32)
    ones = jnp.ones((B, 8), _f32)

    t1, r1 = pl.pallas_call(
        _tc1_body,
        out_shape=[jax.ShapeDtypeStruct((2 * N, 32), _f32),
                   jax.ShapeDtypeStruct((N, 64), _f32)],
    )(x, w1, b1)

    s1, cnt = _sc_layer1(pe, ne, t1, z32, z8, ones)

    u, r2 = pl.pallas_call(
        _tc2_body,
        out_shape=[jax.ShapeDtypeStruct((2 * N, 64), _f32),
                   jax.ShapeDtypeStruct((N, 64), _f32)],
    )(s1, cnt, r1, wp, wn, b2)

    s2 = _sc_layer2(pe, ne, u, z64)

    out = pl.pallas_call(
        _tc3_body,
        out_shape=jax.ShapeDtypeStruct((N, 64), _f32),
    )(s2, cnt, r2)
    return out


# speedup vs baseline: 20.3197x; 1.0342x over previous
import functools

import jax
import jax.numpy as jnp
from jax import lax
from jax.experimental import pallas as pl
from jax.experimental.pallas import tpu as pltpu
from jax.experimental.pallas import tpu_sc as plsc

N = 10000
NP = 10240
E = 160000
NT = 16
CH = 80
B = 125
RPT = NP // NT

_f32 = jnp.float32
_mesh = plsc.VectorSubcoreMesh(core_axis_name="c", subcore_axis_name="s")


NBUF = 8
SLACK = 2


def _agg_pipeline(c, s, PE, NE, T, idx_s, idx_d, rows, acc, gsems, ssems):
    row0 = s * CH

    @pl.when(c == 0)
    def _():
        pltpu.sync_copy(PE.at[0, pl.ds(row0, CH)], idx_s)
        pltpu.sync_copy(PE.at[1, pl.ds(row0, CH)], idx_d)

    @pl.when(c == 1)
    def _():
        pltpu.sync_copy(NE.at[0, pl.ds(row0, CH)], idx_s)
        pltpu.sync_copy(NE.at[1, pl.ds(row0, CH)], idx_d)

    plsc.subcore_barrier()

    for b in range(NBUF):
        pltpu.async_copy(T.at[idx_s.at[b]], rows.at[b], gsems.at[b])

    def group(g, carry):
        for b in range(NBUF):
            j = g * NBUF + b
            pltpu.make_async_copy(T.at[idx_s.at[j]], rows.at[b],
                                  gsems.at[b]).wait()
            pltpu.async_copy(rows.at[b], acc.at[idx_d.at[j]], ssems.at[b],
                             add=True)
            r = j + NBUF - SLACK
            rb = (b + NBUF - SLACK) % NBUF

            @pl.when((r >= NBUF) & (r < CH))
            def _():
                pltpu.make_async_copy(rows.at[rb], acc.at[idx_d.at[0]],
                                      ssems.at[rb]).wait()
                pltpu.async_copy(T.at[idx_s.at[r]], rows.at[rb], gsems.at[rb])

        return carry

    lax.fori_loop(0, CH // NBUF, group, 0)
    for b in range(NBUF):
        pltpu.make_async_copy(rows.at[b], acc.at[idx_d.at[0]],
                              ssems.at[b]).wait()


def _sc_counts_body(PE, NE, Z8, ONES, outC, idx_d, ones_v, accc, osem):
    c = lax.axis_index("c")
    s = lax.axis_index("s")
    row0 = s * CH

    @pl.when(c == 0)
    def _():
        pltpu.sync_copy(PE.at[1, pl.ds(row0, CH)], idx_d)

    @pl.when(c == 1)
    def _():
        pltpu.sync_copy(NE.at[1, pl.ds(row0, CH)], idx_d)

    pltpu.sync_copy(ONES, ones_v)
    r0 = s * RPT
    pltpu.sync_copy(Z8.at[pl.ds(r0, RPT)], accc.at[pl.ds(r0, RPT)])
    plsc.subcore_barrier()

    def fire(j, carry):
        pltpu.async_copy(ones_v, accc.at[idx_d.at[j]], osem, add=True)
        return carry

    lax.fori_loop(0, CH, fire, 0)

    def drain(j, carry):
        pltpu.make_async_copy(ones_v, accc.at[idx_d.at[0]], osem).wait()
        return carry

    lax.fori_loop(0, CH, drain, 0)
    plsc.subcore_barrier()
    o0 = c * NP + r0
    pltpu.sync_copy(accc.at[pl.ds(r0, RPT)], outC.at[pl.ds(o0, RPT)])


_sc_counts = functools.partial(
    pl.kernel,
    out_type=jax.ShapeDtypeStruct((2 * NP, 8), _f32),
    mesh=_mesh,
    scratch_types=(pltpu.VMEM((CH, B), jnp.int32),
                   pltpu.VMEM((B, 8), _f32),
                   pltpu.VMEM_SHARED((NP, 8), _f32),
                   pltpu.SemaphoreType.DMA),
    compiler_params=pltpu.CompilerParams(use_tc_tiling_on_sc=False),
)(_sc_counts_body)


def _sc_agg32_body(PE, NE, T, Z32, outS, idx_s, idx_d, rows, acc,
                   gsems, ssems):
    c = lax.axis_index("c")
    s = lax.axis_index("s")
    r0 = s * RPT
    pltpu.sync_copy(Z32.at[pl.ds(r0, RPT)], acc.at[pl.ds(r0, RPT)])
    _agg_pipeline(c, s, PE, NE, T, idx_s, idx_d, rows, acc, gsems, ssems)
    plsc.subcore_barrier()
    o0 = c * NP + r0
    pltpu.sync_copy(acc.at[pl.ds(r0, RPT)], outS.at[pl.ds(o0, RPT)])


_sc_layer1 = functools.partial(
    pl.kernel,
    out_type=jax.ShapeDtypeStruct((2 * NP, 32), _f32),
    mesh=_mesh,
    scratch_types=(pltpu.VMEM((CH, B), jnp.int32),
                   pltpu.VMEM((CH, B), jnp.int32),
                   pltpu.VMEM((NBUF, B, 32), _f32),
                   pltpu.VMEM_SHARED((NP, 32), _f32),
                   pltpu.SemaphoreType.DMA((NBUF,)),
                   pltpu.SemaphoreType.DMA((NBUF,))),
    compiler_params=pltpu.CompilerParams(use_tc_tiling_on_sc=False),
)(_sc_agg32_body)


def _sc_agg_body(PE, NE, T, Z64, outS, idx_s, idx_d, rows, acc, gsems, ssems):
    c = lax.axis_index("c")
    s = lax.axis_index("s")
    r0 = s * RPT
    pltpu.sync_copy(Z64.at[pl.ds(r0, RPT)], acc.at[pl.ds(r0, RPT)])
    _agg_pipeline(c, s, PE, NE, T, idx_s, idx_d, rows, acc, gsems, ssems)
    plsc.subcore_barrier()
    o0 = c * NP + r0
    pltpu.sync_copy(acc.at[pl.ds(r0, RPT)], outS.at[pl.ds(o0, RPT)])


_sc_layer2 = functools.partial(
    pl.kernel,
    out_type=jax.ShapeDtypeStruct((2 * NP, 64), _f32),
    mesh=_mesh,
    scratch_types=(pltpu.VMEM((CH, B), jnp.int32),
                   pltpu.VMEM((CH, B), jnp.int32),
                   pltpu.VMEM((NBUF, B, 64), _f32),
                   pltpu.VMEM_SHARED((NP, 64), _f32),
                   pltpu.SemaphoreType.DMA((NBUF,)),
                   pltpu.SemaphoreType.DMA((NBUF,))),
    compiler_params=pltpu.CompilerParams(use_tc_tiling_on_sc=False),
)(_sc_agg_body)


def _tc1_body(x_ref, w_ref, b_ref, t_ref, r_ref):
    m = jnp.dot(x_ref[...], w_ref[...], preferred_element_type=_f32)
    t_ref[0:N, :] = m[:, 0:32]
    t_ref[N:2 * N, :] = m[:, 32:64]
    r_ref[...] = m[:, 64:128] + b_ref[...]


def _tc2_body(s1_ref, c_ref, r1_ref, wp_ref, wn_ref, b_ref, u_ref, r2_ref):
    cp = jnp.maximum(c_ref[0:N, 0:1], 1.0)
    cn = jnp.maximum(c_ref[NP:NP + N, 0:1], 1.0)
    zp = jnp.maximum(r1_ref[:, 0:32] + s1_ref[0:N, :] / cp, 0.0)
    zn = jnp.maximum(r1_ref[:, 32:64] + s1_ref[NP:NP + N, :] / cn, 0.0)
    p = jnp.dot(zp, wp_ref[...], preferred_element_type=_f32)
    q = jnp.dot(zn, wn_ref[...], preferred_element_type=_f32)
    u_ref[0:N, 0:32] = p[:, 0:32]
    u_ref[0:N, 32:64] = q[:, 0:32]
    u_ref[N:2 * N, 0:32] = q[:, 32:64]
    u_ref[N:2 * N, 32:64] = p[:, 32:64]
    r2_ref[:, 0:32] = p[:, 64:96] + b_ref[:, 0:32]
    r2_ref[:, 32:64] = q[:, 64:96] + b_ref[:, 32:64]


def _tc3_body(s2_ref, c_ref, r2_ref, out_ref):
    cp = jnp.maximum(c_ref[0:N, 0:1], 1.0)
    cn = jnp.maximum(c_ref[NP:NP + N, 0:1], 1.0)
    ap = s2_ref[0:N, :] / cp
    an = s2_ref[NP:NP + N, :] / cn
    val = jnp.maximum(r2_ref[...] + ap + an, 0.0)
    out_ref[...] = val.T


def kernel(x, pos_edge_index, neg_edge_index,
           c1_Wpl, c1_Wpr, c1_bpr, c1_Wnl, c1_Wnr, c1_bnr,
           c2_Wpl, c2_Wpr, c2_bpr, c2_Wnl, c2_Wnr, c2_bnr):
    w1 = jnp.concatenate([c1_Wpl.T, c1_Wnl.T, c1_Wpr.T, c1_Wnr.T], axis=1)
    b1 = jnp.concatenate([c1_bpr, c1_bnr]).reshape(1, 64)
    wp = jnp.concatenate([c2_Wpl[:, :32].T, c2_Wnl[:, 32:].T, c2_Wpr.T], axis=1)
    wn = jnp.concatenate([c2_Wnl[:, :32].T, c2_Wpl[:, 32:].T, c2_Wnr.T], axis=1)
    b2 = jnp.concatenate([c2_bpr, c2_bnr]).reshape(1, 64)

    pe = pos_edge_index.reshape(2, NT * CH, B)
    ne = (neg_edge_index.reshape(2, NT * CH, B)
          + jnp.array([N, 0], jnp.int32).reshape(2, 1, 1))

    z32 = jnp.zeros((NP, 32), _f32)
    z8 = jnp.zeros((NP, 8), _f32)
    z64 = jnp.zeros((NP, 64), _f32)
    ones = jnp.ones((B, 8), _f32)

    t1, r1 = pl.pallas_call(
        _tc1_body,
        out_shape=[jax.ShapeDtypeStruct((2 * N, 32), _f32),
                   jax.ShapeDtypeStruct((N, 64), _f32)],
    )(x, w1, b1)

    cnt = _sc_counts(pe, ne, z8, ones)
    s1 = _sc_layer1(pe, ne, t1, z32)

    u, r2 = pl.pallas_call(
        _tc2_body,
        out_shape=[jax.ShapeDtypeStruct((2 * N, 64), _f32),
                   jax.ShapeDtypeStruct((N, 64), _f32)],
    )(s1, cnt, r1, wp, wn, b2)

    s2 = _sc_layer2(pe, ne, u, z64)

    out_t = pl.pallas_call(
        _tc3_body,
        out_shape=jax.ShapeDtypeStruct((64, N), _f32),
    )(s2, cnt, r2)
    return out_t.T


# speedup vs baseline: 22.4166x; 1.1032x over previous
import functools

import jax
import jax.numpy as jnp
from jax import lax
from jax.experimental import pallas as pl
from jax.experimental.pallas import tpu as pltpu
from jax.experimental.pallas import tpu_sc as plsc

N = 10000
NP = 10240
E = 160000
NT = 16
CH = 80
B = 125
RPT = NP // NT

_f32 = jnp.float32
_mesh = plsc.VectorSubcoreMesh(core_axis_name="c", subcore_axis_name="s")


NBUF = 8
SLACK = 2


def _agg_pipeline(c, s, PE, NE, T, idx_s, idx_d, rows, acc, gsems, ssems):
    row0 = s * CH

    @pl.when(c == 0)
    def _():
        pltpu.sync_copy(PE.at[0, pl.ds(row0, CH)], idx_s)
        pltpu.sync_copy(PE.at[1, pl.ds(row0, CH)], idx_d)

    @pl.when(c == 1)
    def _():
        pltpu.sync_copy(NE.at[0, pl.ds(row0, CH)], idx_s)
        pltpu.sync_copy(NE.at[1, pl.ds(row0, CH)], idx_d)

    plsc.subcore_barrier()

    for b in range(NBUF):
        pltpu.async_copy(T.at[idx_s.at[b]], rows.at[b], gsems.at[b])

    def group(g, carry):
        for b in range(NBUF):
            j = g * NBUF + b
            pltpu.make_async_copy(T.at[idx_s.at[j]], rows.at[b],
                                  gsems.at[b]).wait()
            pltpu.async_copy(rows.at[b], acc.at[idx_d.at[j]], ssems.at[b],
                             add=True)
            r = j + NBUF - SLACK
            rb = (b + NBUF - SLACK) % NBUF

            @pl.when((r >= NBUF) & (r < CH))
            def _():
                pltpu.make_async_copy(rows.at[rb], acc.at[idx_d.at[0]],
                                      ssems.at[rb]).wait()
                pltpu.async_copy(T.at[idx_s.at[r]], rows.at[rb], gsems.at[rb])

        return carry

    lax.fori_loop(0, CH // NBUF, group, 0)
    for b in range(NBUF):
        pltpu.make_async_copy(rows.at[b], acc.at[idx_d.at[0]],
                              ssems.at[b]).wait()


def _sc_counts_body(PE, NE, Z8, ONES, outC, idx_d, ones_v, accc, osem):
    c = lax.axis_index("c")
    s = lax.axis_index("s")
    row0 = s * CH

    @pl.when(c == 0)
    def _():
        pltpu.sync_copy(PE.at[1, pl.ds(row0, CH)], idx_d)

    @pl.when(c == 1)
    def _():
        pltpu.sync_copy(NE.at[1, pl.ds(row0, CH)], idx_d)

    pltpu.sync_copy(ONES, ones_v)
    r0 = s * RPT
    pltpu.sync_copy(Z8.at[pl.ds(r0, RPT)], accc.at[pl.ds(r0, RPT)])
    plsc.subcore_barrier()

    def fire(j, carry):
        pltpu.async_copy(ones_v, accc.at[idx_d.at[j]], osem, add=True)
        return carry

    lax.fori_loop(0, CH, fire, 0)

    def drain(j, carry):
        pltpu.make_async_copy(ones_v, accc.at[idx_d.at[0]], osem).wait()
        return carry

    lax.fori_loop(0, CH, drain, 0)
    plsc.subcore_barrier()
    o0 = c * NP + r0
    pltpu.sync_copy(accc.at[pl.ds(r0, RPT)], outC.at[pl.ds(o0, RPT)])


_sc_counts = functools.partial(
    pl.kernel,
    out_type=jax.ShapeDtypeStruct((2 * NP, 8), _f32),
    mesh=_mesh,
    scratch_types=(pltpu.VMEM((CH, B), jnp.int32),
                   pltpu.VMEM((B, 8), _f32),
                   pltpu.VMEM_SHARED((NP, 8), _f32),
                   pltpu.SemaphoreType.DMA),
    compiler_params=pltpu.CompilerParams(use_tc_tiling_on_sc=False),
)(_sc_counts_body)


def _sc_agg32_body(PE, NE, T, Z32, outS, idx_s, idx_d, rows, acc,
                   gsems, ssems):
    c = lax.axis_index("c")
    s = lax.axis_index("s")
    r0 = s * RPT
    pltpu.sync_copy(Z32.at[pl.ds(r0, RPT)], acc.at[pl.ds(r0, RPT)])
    _agg_pipeline(c, s, PE, NE, T, idx_s, idx_d, rows, acc, gsems, ssems)
    plsc.subcore_barrier()
    o0 = c * NP + r0
    pltpu.sync_copy(acc.at[pl.ds(r0, RPT)], outS.at[pl.ds(o0, RPT)])


_sc_layer1 = functools.partial(
    pl.kernel,
    out_type=jax.ShapeDtypeStruct((2 * NP, 32), _f32),
    mesh=_mesh,
    scratch_types=(pltpu.VMEM((CH, B), jnp.int32),
                   pltpu.VMEM((CH, B), jnp.int32),
                   pltpu.VMEM((NBUF, B, 32), _f32),
                   pltpu.VMEM_SHARED((NP, 32), _f32),
                   pltpu.SemaphoreType.DMA((NBUF,)),
                   pltpu.SemaphoreType.DMA((NBUF,))),
    compiler_params=pltpu.CompilerParams(use_tc_tiling_on_sc=False),
)(_sc_agg32_body)


def _sc_agg_body(PE, NE, T, Z64, outS, idx_s, idx_d, rows, acc, gsems, ssems):
    c = lax.axis_index("c")
    s = lax.axis_index("s")
    r0 = s * RPT
    pltpu.sync_copy(Z64.at[pl.ds(r0, RPT)], acc.at[pl.ds(r0, RPT)])
    _agg_pipeline(c, s, PE, NE, T, idx_s, idx_d, rows, acc, gsems, ssems)
    plsc.subcore_barrier()
    o0 = c * NP + r0
    pltpu.sync_copy(acc.at[pl.ds(r0, RPT)], outS.at[pl.ds(o0, RPT)])


_sc_layer2 = functools.partial(
    pl.kernel,
    out_type=jax.ShapeDtypeStruct((2 * NP, 64), _f32),
    mesh=_mesh,
    scratch_types=(pltpu.VMEM((CH, B), jnp.int32),
                   pltpu.VMEM((CH, B), jnp.int32),
                   pltpu.VMEM((NBUF, B, 64), _f32),
                   pltpu.VMEM_SHARED((NP, 64), _f32),
                   pltpu.SemaphoreType.DMA((NBUF,)),
                   pltpu.SemaphoreType.DMA((NBUF,))),
    compiler_params=pltpu.CompilerParams(use_tc_tiling_on_sc=False),
)(_sc_agg_body)


def _tc1_body(x_ref, w_ref, b_ref, t_ref, r_ref):
    m = jnp.dot(x_ref[...], w_ref[...], preferred_element_type=_f32)
    t_ref[...] = m[:, 0:64]
    r_ref[...] = m[:, 64:128] + b_ref[...]


def _tc2_body(s1_ref, c_ref, r1_ref, w_ref, b_ref, u_ref, r2_ref):
    cp = jnp.maximum(c_ref[0:N, 0:1], 1.0)
    cn = jnp.maximum(c_ref[NP:NP + N, 0:1], 1.0)
    zp = jnp.maximum(r1_ref[:, 0:32] + s1_ref[0:N, :] / cp, 0.0)
    zn = jnp.maximum(r1_ref[:, 32:64] + s1_ref[NP:NP + N, :] / cn, 0.0)
    z = jnp.concatenate([zp, zn], axis=1)
    m = jnp.dot(z, w_ref[...], preferred_element_type=_f32)
    u_ref[...] = m[:, 0:128]
    r2_ref[...] = m[:, 128:192] + b_ref[...]


def _tc3_body(s2_ref, c_ref, r2_ref, out_ref):
    cp = jnp.maximum(c_ref[0:N, 0:1], 1.0)
    cn = jnp.maximum(c_ref[NP:NP + N, 0:1], 1.0)
    ap = s2_ref[0:N, :] / cp
    an = s2_ref[NP:NP + N, :] / cn
    val = jnp.maximum(r2_ref[...] + ap + an, 0.0)
    out_ref[...] = val.T


def kernel(x, pos_edge_index, neg_edge_index,
           c1_Wpl, c1_Wpr, c1_bpr, c1_Wnl, c1_Wnr, c1_bnr,
           c2_Wpl, c2_Wpr, c2_bpr, c2_Wnl, c2_Wnr, c2_bnr):
    w1 = jnp.concatenate([c1_Wpl.T, c1_Wnl.T, c1_Wpr.T, c1_Wnr.T], axis=1)
    b1 = jnp.concatenate([c1_bpr, c1_bnr]).reshape(1, 64)
    zero = jnp.zeros((32, 32), _f32)
    w2 = jnp.concatenate([
        jnp.concatenate([c2_Wpl[:, :32].T, zero, zero, c2_Wnl[:, 32:].T,
                         c2_Wpr.T, zero], axis=1),
        jnp.concatenate([zero, c2_Wnl[:, :32].T, c2_Wpl[:, 32:].T, zero,
                         zero, c2_Wnr.T], axis=1),
    ], axis=0)
    b2 = jnp.concatenate([c2_bpr, c2_bnr]).reshape(1, 64)

    scale = jnp.array([2, 1], jnp.int32).reshape(2, 1, 1)
    pe = pos_edge_index.reshape(2, NT * CH, B) * scale
    ne = (neg_edge_index.reshape(2, NT * CH, B) * scale
          + jnp.array([1, 0], jnp.int32).reshape(2, 1, 1))

    z32 = jnp.zeros((NP, 32), _f32)
    z8 = jnp.zeros((NP, 8), _f32)
    z64 = jnp.zeros((NP, 64), _f32)
    ones = jnp.ones((B, 8), _f32)

    cnt = _sc_counts(pe, ne, z8, ones)

    t1, r1 = pl.pallas_call(
        _tc1_body,
        out_shape=[jax.ShapeDtypeStruct((N, 64), _f32),
                   jax.ShapeDtypeStruct((N, 64), _f32)],
    )(x, w1, b1)

    s1 = _sc_layer1(pe, ne, t1.reshape(2 * N, 32), z32)

    u, r2 = pl.pallas_call(
        _tc2_body,
        out_shape=[jax.ShapeDtypeStruct((N, 128), _f32),
                   jax.ShapeDtypeStruct((N, 64), _f32)],
    )(s1, cnt, r1, w2, b2)

    s2 = _sc_layer2(pe, ne, u.reshape(2 * N, 64), z64)

    out_t = pl.pallas_call(
        _tc3_body,
        out_shape=jax.ShapeDtypeStruct((64, N), _f32),
    )(s2, cnt, r2)
    return out_t.T


# speedup vs baseline: 22.5851x; 1.0075x over previous
import functools

import jax
import jax.numpy as jnp
from jax import lax
from jax.experimental import pallas as pl
from jax.experimental.pallas import tpu as pltpu
from jax.experimental.pallas import tpu_sc as plsc

N = 10000
NP = 10240
E = 160000
NT = 16
CH = 80
B = 125
RPT = NP // NT

_f32 = jnp.float32
_mesh = plsc.VectorSubcoreMesh(core_axis_name="c", subcore_axis_name="s")


NBUF = 8
SLACK = 4


def _agg_pipeline(c, s, PE, NE, T, idx_s, idx_d, rows, acc, gsems, ssems):
    row0 = s * CH

    @pl.when(c == 0)
    def _():
        pltpu.sync_copy(PE.at[0, pl.ds(row0, CH)], idx_s)
        pltpu.sync_copy(PE.at[1, pl.ds(row0, CH)], idx_d)

    @pl.when(c == 1)
    def _():
        pltpu.sync_copy(NE.at[0, pl.ds(row0, CH)], idx_s)
        pltpu.sync_copy(NE.at[1, pl.ds(row0, CH)], idx_d)

    plsc.subcore_barrier()

    for b in range(NBUF):
        pltpu.async_copy(T.at[idx_s.at[b]], rows.at[b], gsems.at[b])

    def group(g, carry):
        for b in range(NBUF):
            j = g * NBUF + b
            pltpu.make_async_copy(T.at[idx_s.at[j]], rows.at[b],
                                  gsems.at[b]).wait()
            pltpu.async_copy(rows.at[b], acc.at[idx_d.at[j]], ssems.at[b],
                             add=True)
            r = j + NBUF - SLACK
            rb = (b + NBUF - SLACK) % NBUF

            @pl.when((r >= NBUF) & (r < CH))
            def _():
                pltpu.make_async_copy(rows.at[rb], acc.at[idx_d.at[0]],
                                      ssems.at[rb]).wait()
                pltpu.async_copy(T.at[idx_s.at[r]], rows.at[rb], gsems.at[rb])

        return carry

    lax.fori_loop(0, CH // NBUF, group, 0)
    for b in range(NBUF):
        pltpu.make_async_copy(rows.at[b], acc.at[idx_d.at[0]],
                              ssems.at[b]).wait()


def _sc_counts_body(PE, NE, Z8, ONES, outC, idx_d, ones_v, accc, osem):
    c = lax.axis_index("c")
    s = lax.axis_index("s")
    row0 = s * CH

    @pl.when(c == 0)
    def _():
        pltpu.sync_copy(PE.at[1, pl.ds(row0, CH)], idx_d)

    @pl.when(c == 1)
    def _():
        pltpu.sync_copy(NE.at[1, pl.ds(row0, CH)], idx_d)

    pltpu.sync_copy(ONES, ones_v)
    r0 = s * RPT
    pltpu.sync_copy(Z8.at[pl.ds(r0, RPT)], accc.at[pl.ds(r0, RPT)])
    plsc.subcore_barrier()

    def fire(j, carry):
        pltpu.async_copy(ones_v, accc.at[idx_d.at[j]], osem, add=True)
        return carry

    lax.fori_loop(0, CH, fire, 0)

    def drain(j, carry):
        pltpu.make_async_copy(ones_v, accc.at[idx_d.at[0]], osem).wait()
        return carry

    lax.fori_loop(0, CH, drain, 0)
    plsc.subcore_barrier()
    o0 = c * NP + r0
    pltpu.sync_copy(accc.at[pl.ds(r0, RPT)], outC.at[pl.ds(o0, RPT)])


_sc_counts = functools.partial(
    pl.kernel,
    out_type=jax.ShapeDtypeStruct((2 * NP, 8), _f32),
    mesh=_mesh,
    scratch_types=(pltpu.VMEM((CH, B), jnp.int32),
                   pltpu.VMEM((B, 8), _f32),
                   pltpu.VMEM_SHARED((NP, 8), _f32),
                   pltpu.SemaphoreType.DMA),
    compiler_params=pltpu.CompilerParams(use_tc_tiling_on_sc=False),
)(_sc_counts_body)


def _sc_agg32_body(PE, NE, T, Z32, CNT, outS, idx_s, idx_d, rows, acc,
                   gsems, ssems):
    del CNT
    c = lax.axis_index("c")
    s = lax.axis_index("s")
    r0 = s * RPT
    pltpu.sync_copy(Z32.at[pl.ds(r0, RPT)], acc.at[pl.ds(r0, RPT)])
    _agg_pipeline(c, s, PE, NE, T, idx_s, idx_d, rows, acc, gsems, ssems)
    plsc.subcore_barrier()
    o0 = c * NP + r0
    pltpu.sync_copy(acc.at[pl.ds(r0, RPT)], outS.at[pl.ds(o0, RPT)])


_sc_layer1 = functools.partial(
    pl.kernel,
    out_type=jax.ShapeDtypeStruct((2 * NP, 32), _f32),
    mesh=_mesh,
    scratch_types=(pltpu.VMEM((CH, B), jnp.int32),
                   pltpu.VMEM((CH, B), jnp.int32),
                   pltpu.VMEM((NBUF, B, 32), _f32),
                   pltpu.VMEM_SHARED((NP, 32), _f32),
                   pltpu.SemaphoreType.DMA((NBUF,)),
                   pltpu.SemaphoreType.DMA((NBUF,))),
    compiler_params=pltpu.CompilerParams(use_tc_tiling_on_sc=False),
)(_sc_agg32_body)


def _sc_agg_body(PE, NE, T, Z64, outS, idx_s, idx_d, rows, acc, gsems, ssems):
    c = lax.axis_index("c")
    s = lax.axis_index("s")
    r0 = s * RPT
    pltpu.sync_copy(Z64.at[pl.ds(r0, RPT)], acc.at[pl.ds(r0, RPT)])
    _agg_pipeline(c, s, PE, NE, T, idx_s, idx_d, rows, acc, gsems, ssems)
    plsc.subcore_barrier()
    o0 = c * NP + r0
    pltpu.sync_copy(acc.at[pl.ds(r0, RPT)], outS.at[pl.ds(o0, RPT)])


_sc_layer2 = functools.partial(
    pl.kernel,
    out_type=jax.ShapeDtypeStruct((2 * NP, 64), _f32),
    mesh=_mesh,
    scratch_types=(pltpu.VMEM((CH, B), jnp.int32),
                   pltpu.VMEM((CH, B), jnp.int32),
                   pltpu.VMEM((NBUF, B, 64), _f32),
                   pltpu.VMEM_SHARED((NP, 64), _f32),
                   pltpu.SemaphoreType.DMA((NBUF,)),
                   pltpu.SemaphoreType.DMA((NBUF,))),
    compiler_params=pltpu.CompilerParams(use_tc_tiling_on_sc=False),
)(_sc_agg_body)


def _tc1_body(x_ref, w_ref, b_ref, t_ref, r_ref):
    m = jnp.dot(x_ref[...], w_ref[...], preferred_element_type=_f32)
    t_ref[...] = m[:, 0:64]
    r_ref[...] = m[:, 64:128] + b_ref[...]


def _tc2_body(s1_ref, c_ref, r1_ref, w_ref, b_ref, u_ref, r2_ref):
    cp = jnp.maximum(c_ref[0:N, 0:1], 1.0)
    cn = jnp.maximum(c_ref[NP:NP + N, 0:1], 1.0)
    zp = jnp.maximum(r1_ref[:, 0:32] + s1_ref[0:N, :] / cp, 0.0)
    zn = jnp.maximum(r1_ref[:, 32:64] + s1_ref[NP:NP + N, :] / cn, 0.0)
    z = jnp.concatenate([zp, zn], axis=1)
    m = jnp.dot(z, w_ref[...], preferred_element_type=_f32)
    u_ref[...] = m[:, 0:128]
    r2_ref[...] = m[:, 128:192] + b_ref[...]


def _tc3_body(s2_ref, c_ref, r2_ref, out_ref):
    cp = jnp.maximum(c_ref[0:N, 0:1], 1.0)
    cn = jnp.maximum(c_ref[NP:NP + N, 0:1], 1.0)
    ap = s2_ref[0:N, :] / cp
    an = s2_ref[NP:NP + N, :] / cn
    val = jnp.maximum(r2_ref[...] + ap + an, 0.0)
    out_ref[...] = val.T


def kernel(x, pos_edge_index, neg_edge_index,
           c1_Wpl, c1_Wpr, c1_bpr, c1_Wnl, c1_Wnr, c1_bnr,
           c2_Wpl, c2_Wpr, c2_bpr, c2_Wnl, c2_Wnr, c2_bnr):
    w1 = jnp.concatenate([c1_Wpl.T, c1_Wnl.T, c1_Wpr.T, c1_Wnr.T], axis=1)
    b1 = jnp.concatenate([c1_bpr, c1_bnr]).reshape(1, 64)
    zero = jnp.zeros((32, 32), _f32)
    w2 = jnp.concatenate([
        jnp.concatenate([c2_Wpl[:, :32].T, zero, zero, c2_Wnl[:, 32:].T,
                         c2_Wpr.T, zero], axis=1),
        jnp.concatenate([zero, c2_Wnl[:, :32].T, c2_Wpl[:, 32:].T, zero,
                         zero, c2_Wnr.T], axis=1),
    ], axis=0)
    b2 = jnp.concatenate([c2_bpr, c2_bnr]).reshape(1, 64)

    scale = jnp.array([2, 1], jnp.int32).reshape(2, 1, 1)
    pe = pos_edge_index.reshape(2, NT * CH, B) * scale
    ne = (neg_edge_index.reshape(2, NT * CH, B) * scale
          + jnp.array([1, 0], jnp.int32).reshape(2, 1, 1))

    z32 = jnp.zeros((NP, 32), _f32)
    z8 = jnp.zeros((NP, 8), _f32)
    z64 = jnp.zeros((NP, 64), _f32)
    ones = jnp.ones((B, 8), _f32)

    cnt = _sc_counts(pe, ne, z8, ones)

    t1, r1 = pl.pallas_call(
        _tc1_body,
        out_shape=[jax.ShapeDtypeStruct((N, 64), _f32),
                   jax.ShapeDtypeStruct((N, 64), _f32)],
    )(x, w1, b1)

    s1 = _sc_layer1(pe, ne, t1.reshape(2 * N, 32), z32, cnt)

    u, r2 = pl.pallas_call(
        _tc2_body,
        out_shape=[jax.ShapeDtypeStruct((N, 128), _f32),
                   jax.ShapeDtypeStruct((N, 64), _f32)],
    )(s1, cnt, r1, w2, b2)

    s2 = _sc_layer2(pe, ne, u.reshape(2 * N, 64), z64)

    out_t = pl.pallas_call(
        _tc3_body,
        out_shape=jax.ShapeDtypeStruct((64, N), _f32),
    )(s2, cnt, r2)
    return out_t.T


# speedup vs baseline: 23.6689x; 1.0480x over previous
import functools

import jax
import jax.numpy as jnp
from jax import lax
from jax.experimental import pallas as pl
from jax.experimental.pallas import tpu as pltpu
from jax.experimental.pallas import tpu_sc as plsc

N = 10000
NP = 10240
E = 160000
NT = 16
CH = 80
B = 125
RPT = NP // NT

_f32 = jnp.float32
_mesh = plsc.VectorSubcoreMesh(core_axis_name="c", subcore_axis_name="s")


NBUF = 8
SLACK = 2


def _agg_pipeline(c, s, PE, NE, T, idx_s, idx_d, rows, acc, gsems, ssems):
    row0 = s * CH

    @pl.when(c == 0)
    def _():
        pltpu.sync_copy(PE.at[0, pl.ds(row0, CH)], idx_s)
        pltpu.sync_copy(PE.at[1, pl.ds(row0, CH)], idx_d)

    @pl.when(c == 1)
    def _():
        pltpu.sync_copy(NE.at[0, pl.ds(row0, CH)], idx_s)
        pltpu.sync_copy(NE.at[1, pl.ds(row0, CH)], idx_d)

    plsc.subcore_barrier()

    for b in range(NBUF):
        pltpu.async_copy(T.at[idx_s.at[b]], rows.at[b], gsems.at[b])

    def group(g, carry):
        for b in range(NBUF):
            j = g * NBUF + b
            pltpu.make_async_copy(T.at[idx_s.at[j]], rows.at[b],
                                  gsems.at[b]).wait()
            pltpu.async_copy(rows.at[b], acc.at[idx_d.at[j]], ssems.at[b],
                             add=True)
            r = j + NBUF - SLACK
            rb = (b + NBUF - SLACK) % NBUF

            @pl.when((r >= NBUF) & (r < CH))
            def _():
                pltpu.make_async_copy(rows.at[rb], acc.at[idx_d.at[0]],
                                      ssems.at[rb]).wait()
                pltpu.async_copy(T.at[idx_s.at[r]], rows.at[rb], gsems.at[rb])

        return carry

    lax.fori_loop(0, CH // NBUF, group, 0)
    for b in range(NBUF):
        pltpu.make_async_copy(rows.at[b], acc.at[idx_d.at[0]],
                              ssems.at[b]).wait()


def _sc_counts_body(PE, NE, Z8, ONES, outC, idx_d, ones_v, accc, osem):
    c = lax.axis_index("c")
    s = lax.axis_index("s")
    row0 = s * CH

    @pl.when(c == 0)
    def _():
        pltpu.sync_copy(PE.at[1, pl.ds(row0, CH)], idx_d)

    @pl.when(c == 1)
    def _():
        pltpu.sync_copy(NE.at[1, pl.ds(row0, CH)], idx_d)

    pltpu.sync_copy(ONES, ones_v)
    r0 = s * RPT
    pltpu.sync_copy(Z8.at[pl.ds(r0, RPT)], accc.at[pl.ds(r0, RPT)])
    plsc.subcore_barrier()

    def fire(j, carry):
        pltpu.async_copy(ones_v, accc.at[idx_d.at[j]], osem, add=True)
        return carry

    lax.fori_loop(0, CH, fire, 0)

    def drain(j, carry):
        pltpu.make_async_copy(ones_v, accc.at[idx_d.at[0]], osem).wait()
        return carry

    lax.fori_loop(0, CH, drain, 0)
    plsc.subcore_barrier()
    o0 = c * NP + r0
    pltpu.sync_copy(accc.at[pl.ds(r0, RPT)], outC.at[pl.ds(o0, RPT)])


_sc_counts = functools.partial(
    pl.kernel,
    out_type=jax.ShapeDtypeStruct((2 * NP, 8), _f32),
    mesh=_mesh,
    scratch_types=(pltpu.VMEM((CH, B), jnp.int32),
                   pltpu.VMEM((B, 8), _f32),
                   pltpu.VMEM_SHARED((NP, 8), _f32),
                   pltpu.SemaphoreType.DMA),
    compiler_params=pltpu.CompilerParams(use_tc_tiling_on_sc=False),
)(_sc_counts_body)


def _sc_agg32_body(PE, NE, T, Z32, CNT, outS, idx_s, idx_d, rows, acc,
                   gsems, ssems):
    del CNT
    c = lax.axis_index("c")
    s = lax.axis_index("s")
    r0 = s * RPT
    pltpu.sync_copy(Z32.at[pl.ds(r0, RPT)], acc.at[pl.ds(r0, RPT)])
    _agg_pipeline(c, s, PE, NE, T, idx_s, idx_d, rows, acc, gsems, ssems)
    plsc.subcore_barrier()
    o0 = c * NP + r0
    pltpu.sync_copy(acc.at[pl.ds(r0, RPT)], outS.at[pl.ds(o0, RPT)])


_sc_layer1 = functools.partial(
    pl.kernel,
    out_type=jax.ShapeDtypeStruct((2 * NP, 32), _f32),
    mesh=_mesh,
    scratch_types=(pltpu.VMEM((CH, B), jnp.int32),
                   pltpu.VMEM((CH, B), jnp.int32),
                   pltpu.VMEM((NBUF, B, 32), _f32),
                   pltpu.VMEM_SHARED((NP, 32), _f32),
                   pltpu.SemaphoreType.DMA((NBUF,)),
                   pltpu.SemaphoreType.DMA((NBUF,))),
    compiler_params=pltpu.CompilerParams(use_tc_tiling_on_sc=False),
)(_sc_agg32_body)


def _sc_agg_body(PE, NE, T, Z64, outS, idx_s, idx_d, rows, acc, gsems, ssems):
    c = lax.axis_index("c")
    s = lax.axis_index("s")
    r0 = s * RPT
    pltpu.sync_copy(Z64.at[pl.ds(r0, RPT)], acc.at[pl.ds(r0, RPT)])
    _agg_pipeline(c, s, PE, NE, T, idx_s, idx_d, rows, acc, gsems, ssems)
    plsc.subcore_barrier()
    o0 = c * NP + r0
    pltpu.sync_copy(acc.at[pl.ds(r0, RPT)], outS.at[pl.ds(o0, RPT)])


_sc_layer2 = functools.partial(
    pl.kernel,
    out_type=jax.ShapeDtypeStruct((2 * NP, 64), _f32),
    mesh=_mesh,
    scratch_types=(pltpu.VMEM((CH, B), jnp.int32),
                   pltpu.VMEM((CH, B), jnp.int32),
                   pltpu.VMEM((NBUF, B, 64), _f32),
                   pltpu.VMEM_SHARED((NP, 64), _f32),
                   pltpu.SemaphoreType.DMA((NBUF,)),
                   pltpu.SemaphoreType.DMA((NBUF,))),
    compiler_params=pltpu.CompilerParams(use_tc_tiling_on_sc=False),
)(_sc_agg_body)


def _tc1_body(x_ref, w_ref, b_ref, t_ref, r_ref):
    m = jnp.dot(x_ref[...], w_ref[...], preferred_element_type=_f32)
    t_ref[...] = m[:, 0:64]
    r_ref[...] = m[:, 64:128] + b_ref[...]


def _tc2_body(s1_ref, c_ref, r1_ref, w_ref, b_ref, u_ref, r2_ref):
    cp = jnp.maximum(c_ref[0:N, 0:1], 1.0)
    cn = jnp.maximum(c_ref[NP:NP + N, 0:1], 1.0)
    zp = jnp.maximum(r1_ref[:, 0:32] + s1_ref[0:N, :] / cp, 0.0)
    zn = jnp.maximum(r1_ref[:, 32:64] + s1_ref[NP:NP + N, :] / cn, 0.0)
    z = jnp.concatenate([zp, zn], axis=1)
    m = jnp.dot(z, w_ref[...], preferred_element_type=_f32)
    u_ref[...] = m[:, 0:128]
    r2_ref[...] = m[:, 128:192] + b_ref[...]


def _tc3_body(s2_ref, c_ref, r2_ref, out_ref):
    cp = jnp.maximum(c_ref[0:N, 0:1], 1.0)
    cn = jnp.maximum(c_ref[NP:NP + N, 0:1], 1.0)
    ap = s2_ref[0:N, :] / cp
    an = s2_ref[NP:NP + N, :] / cn
    val = jnp.maximum(r2_ref[...] + ap + an, 0.0)
    out_ref[...] = val.T


def kernel(x, pos_edge_index, neg_edge_index,
           c1_Wpl, c1_Wpr, c1_bpr, c1_Wnl, c1_Wnr, c1_bnr,
           c2_Wpl, c2_Wpr, c2_bpr, c2_Wnl, c2_Wnr, c2_bnr):
    w1 = jnp.concatenate([c1_Wpl.T, c1_Wnl.T, c1_Wpr.T, c1_Wnr.T], axis=1)
    b1 = jnp.concatenate([c1_bpr, c1_bnr]).reshape(1, 64)
    zero = jnp.zeros((32, 32), _f32)
    w2 = jnp.concatenate([
        jnp.concatenate([c2_Wpl[:, :32].T, zero, zero, c2_Wnl[:, 32:].T,
                         c2_Wpr.T, zero], axis=1),
        jnp.concatenate([zero, c2_Wnl[:, :32].T, c2_Wpl[:, 32:].T, zero,
                         zero, c2_Wnr.T], axis=1),
    ], axis=0)
    b2 = jnp.concatenate([c2_bpr, c2_bnr]).reshape(1, 64)

    scale = jnp.array([2, 1], jnp.int32).reshape(2, 1, 1)
    pe = pos_edge_index.reshape(2, NT * CH, B) * scale
    ne = (neg_edge_index.reshape(2, NT * CH, B) * scale
          + jnp.array([1, 0], jnp.int32).reshape(2, 1, 1))

    z32 = jnp.zeros((NP, 32), _f32)
    z8 = jnp.zeros((NP, 8), _f32)
    z64 = jnp.zeros((NP, 64), _f32)
    ones = jnp.ones((B, 8), _f32)

    cnt = _sc_counts(pe, ne, z8, ones)

    t1, r1 = pl.pallas_call(
        _tc1_body,
        out_shape=[jax.ShapeDtypeStruct((N, 64), _f32),
                   jax.ShapeDtypeStruct((N, 64), _f32)],
    )(x, w1, b1)

    s1 = _sc_layer1(pe, ne, t1.reshape(2 * N, 32), z32, cnt)

    u, r2 = pl.pallas_call(
        _tc2_body,
        out_shape=[jax.ShapeDtypeStruct((N, 128), _f32),
                   jax.ShapeDtypeStruct((N, 64), _f32)],
    )(s1, cnt, r1, w2, b2)

    s2 = _sc_layer2(pe, ne, u.reshape(2 * N, 64), z64)

    out_t = pl.pallas_call(
        _tc3_body,
        out_shape=jax.ShapeDtypeStruct((64, N), _f32),
    )(s2, cnt, r2)
    return out_t.T
